# baseline (speedup 1.0000x reference)
import functools

import jax
import jax.numpy as jnp
from jax import lax
from jax.experimental import pallas as pl
from jax.experimental.pallas import tpu as pltpu
from jax.experimental.pallas import tpu_sc as plsc

N = 10000
E = 320000
D = 128

NC = 2
NS = 16
NW = NC * NS

C = 80
EPW = E // NW
NCHUNK = EPW // C
SS = 2000
SPC = SS // C
NSUP = EPW // SS
NP = 10240
ZR = NP // NS
ZC = 80
NZ = ZR // ZC
H0 = 32
H1 = 48

_PREP_B = 400
_CMB_B = 512


def _prep_body(x_ref, w_ref, fsw_ref, fw2_ref, fb_ref, dia_ref, pre_ref, tab_ref):
    xb = x_ref[...]
    pre_ref[...] = jnp.dot(xb, w_ref[...], preferred_element_type=jnp.float32)
    fpre = jnp.dot(xb, fsw_ref[...], preferred_element_type=jnp.float32)
    gs = jnp.dot(fpre, fw2_ref[...], preferred_element_type=jnp.float32)
    g1 = gs[:, 0:1] + fb_ref[0, 0]
    s2 = gs[:, 1:2]
    ld = jnp.log(dia_ref[...])
    tab_ref[...] = jnp.concatenate([g1, s2, ld], axis=1)


def _combine_body(a_ref, b_ref, o_ref):
    o_ref[...] = jnp.maximum(a_ref[...] + b_ref[...], 0.0)


def _gate_body(tab_hbm, row_hbm, col_hbm, val_hbm, tab_v, row_s, col_s, val_s):
    cid = lax.axis_index("c")
    sid = lax.axis_index("s")
    wid = cid * NS + sid
    pltpu.sync_copy(tab_hbm, tab_v)
    base = wid * EPW

    def _super(s, carry):
        bs = base + s * SS
        pltpu.sync_copy(row_hbm.at[pl.ds(bs, SS)], row_s)
        pltpu.sync_copy(col_hbm.at[pl.ds(bs, SS)], col_s)

        @plsc.parallel_loop(0, SS // 16, step=1, unroll=8)
        def _grp(g):
            off = pl.multiple_of(g * 16, 16)
            rvec = row_s[pl.ds(off, 16)] * 3
            cvec = col_s[pl.ds(off, 16)] * 3
            g1 = plsc.load_gather(tab_v, [rvec])
            s2 = plsc.load_gather(tab_v, [cvec + 1])
            ldr = plsc.load_gather(tab_v, [rvec + 2])
            ldc = plsc.load_gather(tab_v, [cvec + 2])
            p = 1.0 / (1.0 + jnp.exp(-(g1 + s2)))
            val_s[pl.ds(off, 16)] = jnp.exp(-(p * (ldr + ldc)))
        pltpu.sync_copy(val_s, val_hbm.at[pl.ds(bs, SS)])
        return carry

    lax.fori_loop(0, NSUP, _super, 0)


_gate_call = functools.partial(
    pl.kernel,
    out_type=jax.ShapeDtypeStruct((E,), jnp.float32),
    mesh=plsc.VectorSubcoreMesh(core_axis_name="c", subcore_axis_name="s"),
    scratch_types=[
        pltpu.VMEM((3 * N,), jnp.float32),
        pltpu.VMEM((SS,), jnp.int32),
        pltpu.VMEM((SS,), jnp.int32),
        pltpu.VMEM((SS,), jnp.float32),
    ],
    compiler_params=pltpu.CompilerParams(needs_layout_passes=False),
)(_gate_body)


def _agg_body(pre_hbm, row_hbm, col_hbm, val_hbm, out_hbm,
              row_s, col_s, val_s,
              row_a, col_a, val_a, row_b, col_b, val_b,
              rows_a, rows_b, srows0, srows1, rowh0, rowh1,
              acc, sem_a, sem_b, ssem0, ssem1):
    cid = lax.axis_index("c")
    sid = lax.axis_index("s")
    wid = cid * NS + sid
    base = wid * EPW

    zero16 = jnp.zeros((16,), jnp.float32)

    def _zrow(j, carry):
        for k in range(D // 16):
            rows_a[j, pl.ds(k * 16, 16)] = zero16
        return carry

    lax.fori_loop(0, ZC, _zrow, 0)

    def _zcopy(i, carry):
        pltpu.sync_copy(rows_a, acc.at[pl.ds(sid * ZR + i * ZC, ZC)])
        return carry

    lax.fori_loop(0, NZ, _zcopy, 0)
    plsc.subcore_barrier()

    halves = ((0, H0, srows0, rowh0, ssem0), (H0, H1, srows1, rowh1, ssem1))
    for h0, hn, srows_h, rowh_h, ssem_h in halves:
        def _zs(j, carry, srows_h=srows_h):
            for k in range(D // 16):
                srows_h[j, pl.ds(k * 16, 16)] = zero16
            return carry

        lax.fori_loop(0, hn, _zs, 0)
        for g in range(hn // 16):
            rowh_h[pl.ds(g * 16, 16)] = jnp.zeros((16,), jnp.int32)
        pltpu.async_copy(srows_h, acc.at[rowh_h], ssem_h, add=True)

    def _issue(j, row_x, col_x, val_x, rows_x, sem_x):
        q = j // SPC
        r = j - q * SPC

        @pl.when(r == 0)
        def _():
            bs = base + q * SS
            pltpu.sync_copy(row_hbm.at[pl.ds(bs, SS)], row_s)
            pltpu.sync_copy(col_hbm.at[pl.ds(bs, SS)], col_s)
            pltpu.sync_copy(val_hbm.at[pl.ds(bs, SS)], val_s)

        off = pl.multiple_of(r * C, 16)
        for g in range(C // 16):
            row_x[pl.ds(g * 16, 16)] = row_s[pl.ds(off + g * 16, 16)]
            col_x[pl.ds(g * 16, 16)] = col_s[pl.ds(off + g * 16, 16)]
            val_x[pl.ds(g * 16, 16)] = val_s[pl.ds(off + g * 16, 16)]
        pltpu.async_copy(pre_hbm.at[col_x], rows_x, sem_x)

    def _process(row_x, col_x, val_x, rows_x, sem_x):
        pltpu.make_async_copy(pre_hbm.at[col_x], rows_x, sem_x).wait()
        for h0, hn, srows_h, rowh_h, ssem_h in halves:
            pltpu.make_async_copy(srows_h, acc.at[rowh_h], ssem_h).wait()
            for g in range(hn // 16):
                rowh_h[pl.ds(g * 16, 16)] = row_x[pl.ds(h0 + g * 16, 16)]

            @plsc.parallel_loop(0, hn // 16, step=1, unroll=2)
            def _scale(g, h0=h0, srows_h=srows_h):
                off = pl.multiple_of(g * 16, 16)
                vals = val_x[pl.ds(pl.multiple_of(h0 + g * 16, 16), 16)]
                for l in range(16):
                    jl = off + l
                    sv = vals[l]
                    for k in range(D // 16):
                        srows_h[jl, pl.ds(k * 16, 16)] = (
                            rows_x[h0 + jl, pl.ds(k * 16, 16)] * sv)
            pltpu.async_copy(srows_h, acc.at[rowh_h], ssem_h, add=True)

    bufs_a = (row_a, col_a, val_a, rows_a, sem_a)
    bufs_b = (row_b, col_b, val_b, rows_b, sem_b)

    _issue(0, *bufs_a)

    def _pair(p, carry):
        _issue(2 * p + 1, *bufs_b)
        _process(*bufs_a)
        _issue(2 * p + 2, *bufs_a)
        _process(*bufs_b)
        return carry

    lax.fori_loop(0, (NCHUNK - 1) // 2, _pair, 0)
    _process(*bufs_a)
    for h0, hn, srows_h, rowh_h, ssem_h in halves:
        pltpu.make_async_copy(srows_h, acc.at[rowh_h], ssem_h).wait()
    plsc.subcore_barrier()

    ob = cid * NP + sid * ZR

    def _wb(i, carry):
        pltpu.sync_copy(acc.at[pl.ds(sid * ZR + i * ZC, ZC)], rows_a)
        pltpu.sync_copy(rows_a, out_hbm.at[pl.ds(ob + i * ZC, ZC)])
        return carry

    lax.fori_loop(0, NZ, _wb, 0)


_agg_call = functools.partial(
    pl.kernel,
    out_type=jax.ShapeDtypeStruct((NC * NP, D), jnp.float32),
    mesh=plsc.VectorSubcoreMesh(core_axis_name="c", subcore_axis_name="s"),
    scratch_types=[
        pltpu.VMEM((SS,), jnp.int32),
        pltpu.VMEM((SS,), jnp.int32),
        pltpu.VMEM((SS,), jnp.float32),
        pltpu.VMEM((C,), jnp.int32),
        pltpu.VMEM((C,), jnp.int32),
        pltpu.VMEM((C,), jnp.float32),
        pltpu.VMEM((C,), jnp.int32),
        pltpu.VMEM((C,), jnp.int32),
        pltpu.VMEM((C,), jnp.float32),
        pltpu.VMEM((C, D), jnp.float32),
        pltpu.VMEM((C, D), jnp.float32),
        pltpu.VMEM((H0, D), jnp.float32),
        pltpu.VMEM((H1, D), jnp.float32),
        pltpu.VMEM((H0,), jnp.int32),
        pltpu.VMEM((H1,), jnp.int32),
        pltpu.VMEM_SHARED((NP, D), jnp.float32),
        pltpu.SemaphoreType.DMA,
        pltpu.SemaphoreType.DMA,
        pltpu.SemaphoreType.DMA,
        pltpu.SemaphoreType.DMA,
    ],
    compiler_params=pltpu.CompilerParams(needs_layout_passes=False),
)(_agg_body)


def kernel(x, row, col, dia_adj, W, fsW, fW, fb):
    fw2 = fW.reshape(2, D).transpose(1, 0)
    fb2 = fb.reshape(1, 1)
    dia2 = dia_adj.reshape(N, 1)

    pre, tab = pl.pallas_call(
        _prep_body,
        grid=(N // _PREP_B,),
        in_specs=[
            pl.BlockSpec((_PREP_B, D), lambda i: (i, 0)),
            pl.BlockSpec((D, D), lambda i: (0, 0)),
            pl.BlockSpec((D, D), lambda i: (0, 0)),
            pl.BlockSpec((D, 2), lambda i: (0, 0)),
            pl.BlockSpec((1, 1), lambda i: (0, 0)),
            pl.BlockSpec((_PREP_B, 1), lambda i: (i, 0)),
        ],
        out_specs=[
            pl.BlockSpec((_PREP_B, D), lambda i: (i, 0)),
            pl.BlockSpec((_PREP_B, 3), lambda i: (i, 0)),
        ],
        out_shape=[
            jax.ShapeDtypeStruct((N, D), jnp.float32),
            jax.ShapeDtypeStruct((N, 3), jnp.float32),
        ],
    )(x, W, fsW, fw2, fb2, dia2)

    val = _gate_call(tab.reshape(-1), row, col)
    parts = _agg_call(pre, row, col, val)

    out = pl.pallas_call(
        _combine_body,
        grid=(NP // _CMB_B,),
        in_specs=[
            pl.BlockSpec((_CMB_B, D), lambda i: (i, 0)),
            pl.BlockSpec((_CMB_B, D), lambda i: (NP // _CMB_B + i, 0)),
        ],
        out_specs=pl.BlockSpec((_CMB_B, D), lambda i: (i, 0)),
        out_shape=jax.ShapeDtypeStruct((NP, D), jnp.float32),
    )(parts, parts)
    return out[:N]


# speedup vs baseline: 1.2157x; 1.2157x over previous
import functools

import jax
import jax.numpy as jnp
from jax import lax
from jax.experimental import pallas as pl
from jax.experimental.pallas import tpu as pltpu
from jax.experimental.pallas import tpu_sc as plsc

N = 10000
E = 320000
D = 128

NC = 2
NS = 16
NW = NC * NS

C = 80
EPW = E // NW
NCHUNK = EPW // C
SS = 2000
SPC = SS // C
NSUP = EPW // SS
NP = 10240
ZR = NP // NS
ZC = 80
NZ = ZR // ZC
H0 = 32
H1 = 48

_PREP_B = 400
_CMB_B = 512


def _prep_body(x_ref, w_ref, fsw_ref, fw2_ref, fb_ref, dia_ref, pre_ref, tab_ref):
    xb = x_ref[...]
    pre_ref[...] = jnp.dot(xb, w_ref[...], preferred_element_type=jnp.float32)
    fpre = jnp.dot(xb, fsw_ref[...], preferred_element_type=jnp.float32)
    gs = jnp.dot(fpre, fw2_ref[...], preferred_element_type=jnp.float32)
    g1 = gs[:, 0:1] + fb_ref[0, 0]
    s2 = gs[:, 1:2]
    ld = jnp.log(dia_ref[...])
    tab_ref[...] = jnp.concatenate([g1, s2, ld], axis=1)


def _combine_body(a_ref, b_ref, o_ref):
    o_ref[...] = jnp.maximum(a_ref[...] + b_ref[...], 0.0)


def _edge_body(pre_hbm, tab_hbm, row_hbm, col_hbm, out_hbm, val_hbm,
               row_s, col_s, val_s,
               row_a, col_a, val_a, row_b, col_b, val_b,
               acc, sem_a, sem_b, ssem0, ssem1):
    cid = lax.axis_index("c")
    sid = lax.axis_index("s")
    wid = cid * NS + sid
    base = wid * EPW

    def _gate_phase(tab_v):
        pltpu.sync_copy(tab_hbm, tab_v)

        def _super(s, carry):
            bs = base + s * SS
            pltpu.sync_copy(row_hbm.at[pl.ds(bs, SS)], row_s)
            pltpu.sync_copy(col_hbm.at[pl.ds(bs, SS)], col_s)

            @plsc.parallel_loop(0, SS // 16, step=1, unroll=4)
            def _grp(g):
                off = pl.multiple_of(g * 16, 16)
                rvec = row_s[pl.ds(off, 16)] * 3
                cvec = col_s[pl.ds(off, 16)] * 3
                g1 = plsc.load_gather(tab_v, [rvec])
                s2 = plsc.load_gather(tab_v, [cvec + 1])
                ldr = plsc.load_gather(tab_v, [rvec + 2])
                ldc = plsc.load_gather(tab_v, [cvec + 2])
                p = 1.0 / (1.0 + jnp.exp(-(g1 + s2)))
                val_s[pl.ds(off, 16)] = jnp.exp(-(p * (ldr + ldc)))
            pltpu.sync_copy(val_s, val_hbm.at[pl.ds(bs, SS)])
            return carry

        lax.fori_loop(0, NSUP, _super, 0)

    pl.run_scoped(_gate_phase, pltpu.VMEM((3 * N,), jnp.float32))

    def _agg_phase(rows_a, rows_b, srows0, srows1, rowh0, rowh1):
        zero16 = jnp.zeros((16,), jnp.float32)
        halves = ((0, H0, srows0, rowh0, ssem0), (H0, H1, srows1, rowh1, ssem1))
        for h0, hn, srows_h, rowh_h, ssem_h in halves:
            def _zs(j, carry, srows_h=srows_h):
                for k in range(D // 16):
                    srows_h[j, pl.ds(k * 16, 16)] = zero16
                return carry

            lax.fori_loop(0, hn, _zs, 0)

        def _zcopy(i, carry):
            pltpu.sync_copy(srows0, acc.at[pl.ds(sid * ZR + i * H0, H0)])
            return carry

        lax.fori_loop(0, ZR // H0, _zcopy, 0)
        plsc.subcore_barrier()

        for h0, hn, srows_h, rowh_h, ssem_h in halves:
            for g in range(hn // 16):
                rowh_h[pl.ds(g * 16, 16)] = jnp.zeros((16,), jnp.int32)
            pltpu.async_copy(srows_h, acc.at[rowh_h], ssem_h, add=True)

        def _issue(j, row_x, col_x, val_x, rows_x, sem_x):
            q = j // SPC
            r = j - q * SPC

            @pl.when(r == 0)
            def _():
                bs = base + q * SS
                pltpu.sync_copy(row_hbm.at[pl.ds(bs, SS)], row_s)
                pltpu.sync_copy(col_hbm.at[pl.ds(bs, SS)], col_s)
                pltpu.sync_copy(val_hbm.at[pl.ds(bs, SS)], val_s)

            off = pl.multiple_of(r * C, 16)
            for g in range(C // 16):
                row_x[pl.ds(g * 16, 16)] = row_s[pl.ds(off + g * 16, 16)]
                col_x[pl.ds(g * 16, 16)] = col_s[pl.ds(off + g * 16, 16)]
                val_x[pl.ds(g * 16, 16)] = val_s[pl.ds(off + g * 16, 16)]
            pltpu.async_copy(pre_hbm.at[col_x], rows_x, sem_x)

        def _process(row_x, col_x, val_x, rows_x, sem_x):
            pltpu.make_async_copy(pre_hbm.at[col_x], rows_x, sem_x).wait()
            for h0, hn, srows_h, rowh_h, ssem_h in halves:
                pltpu.make_async_copy(srows_h, acc.at[rowh_h], ssem_h).wait()
                for g in range(hn // 16):
                    rowh_h[pl.ds(g * 16, 16)] = row_x[pl.ds(h0 + g * 16, 16)]

                @plsc.parallel_loop(0, hn // 16, step=1, unroll=1)
                def _scale(g, h0=h0, srows_h=srows_h):
                    off = pl.multiple_of(g * 16, 16)
                    vals = val_x[pl.ds(pl.multiple_of(h0 + g * 16, 16), 16)]
                    for l in range(16):
                        jl = off + l
                        sv = vals[l]
                        for k in range(D // 16):
                            srows_h[jl, pl.ds(k * 16, 16)] = (
                                rows_x[h0 + jl, pl.ds(k * 16, 16)] * sv)
                pltpu.async_copy(srows_h, acc.at[rowh_h], ssem_h, add=True)

        bufs_a = (row_a, col_a, val_a, rows_a, sem_a)
        bufs_b = (row_b, col_b, val_b, rows_b, sem_b)

        _issue(0, *bufs_a)

        def _pair(p, carry):
            _issue(2 * p + 1, *bufs_b)
            _process(*bufs_a)
            _issue(2 * p + 2, *bufs_a)
            _process(*bufs_b)
            return carry

        lax.fori_loop(0, (NCHUNK - 1) // 2, _pair, 0)
        _process(*bufs_a)
        for h0, hn, srows_h, rowh_h, ssem_h in halves:
            pltpu.make_async_copy(srows_h, acc.at[rowh_h], ssem_h).wait()
        plsc.subcore_barrier()

        ob = cid * NP + sid * ZR

        def _wb(i, carry):
            pltpu.sync_copy(acc.at[pl.ds(sid * ZR + i * H0, H0)], srows0)
            pltpu.sync_copy(srows0, out_hbm.at[pl.ds(ob + i * H0, H0)])
            return carry

        lax.fori_loop(0, ZR // H0, _wb, 0)

    pl.run_scoped(
        _agg_phase,
        pltpu.VMEM((C, D), jnp.float32),
        pltpu.VMEM((C, D), jnp.float32),
        pltpu.VMEM((H0, D), jnp.float32),
        pltpu.VMEM((H1, D), jnp.float32),
        pltpu.VMEM((H0,), jnp.int32),
        pltpu.VMEM((H1,), jnp.int32),
    )


_edge_call = functools.partial(
    pl.kernel,
    out_type=(jax.ShapeDtypeStruct((NC * NP, D), jnp.float32),
              jax.ShapeDtypeStruct((E,), jnp.float32)),
    mesh=plsc.VectorSubcoreMesh(core_axis_name="c", subcore_axis_name="s"),
    scratch_types=[
        pltpu.VMEM((SS,), jnp.int32),
        pltpu.VMEM((SS,), jnp.int32),
        pltpu.VMEM((SS,), jnp.float32),
        pltpu.VMEM((C,), jnp.int32),
        pltpu.VMEM((C,), jnp.int32),
        pltpu.VMEM((C,), jnp.float32),
        pltpu.VMEM((C,), jnp.int32),
        pltpu.VMEM((C,), jnp.int32),
        pltpu.VMEM((C,), jnp.float32),
        pltpu.VMEM_SHARED((NP, D), jnp.float32),
        pltpu.SemaphoreType.DMA,
        pltpu.SemaphoreType.DMA,
        pltpu.SemaphoreType.DMA,
        pltpu.SemaphoreType.DMA,
    ],
    compiler_params=pltpu.CompilerParams(needs_layout_passes=False),
)(_edge_body)


def kernel(x, row, col, dia_adj, W, fsW, fW, fb):
    fw2 = fW.reshape(2, D).transpose(1, 0)
    fb2 = fb.reshape(1, 1)
    dia2 = dia_adj.reshape(N, 1)

    pre, tab = pl.pallas_call(
        _prep_body,
        grid=(N // _PREP_B,),
        in_specs=[
            pl.BlockSpec((_PREP_B, D), lambda i: (i, 0)),
            pl.BlockSpec((D, D), lambda i: (0, 0)),
            pl.BlockSpec((D, D), lambda i: (0, 0)),
            pl.BlockSpec((D, 2), lambda i: (0, 0)),
            pl.BlockSpec((1, 1), lambda i: (0, 0)),
            pl.BlockSpec((_PREP_B, 1), lambda i: (i, 0)),
        ],
        out_specs=[
            pl.BlockSpec((_PREP_B, D), lambda i: (i, 0)),
            pl.BlockSpec((_PREP_B, 3), lambda i: (i, 0)),
        ],
        out_shape=[
            jax.ShapeDtypeStruct((N, D), jnp.float32),
            jax.ShapeDtypeStruct((N, 3), jnp.float32),
        ],
    )(x, W, fsW, fw2, fb2, dia2)

    parts, _ = _edge_call(pre, tab.reshape(-1), row, col)

    out = pl.pallas_call(
        _combine_body,
        grid=(NP // _CMB_B,),
        in_specs=[
            pl.BlockSpec((_CMB_B, D), lambda i: (i, 0)),
            pl.BlockSpec((_CMB_B, D), lambda i: (NP // _CMB_B + i, 0)),
        ],
        out_specs=pl.BlockSpec((_CMB_B, D), lambda i: (i, 0)),
        out_shape=jax.ShapeDtypeStruct((NP, D), jnp.float32),
    )(parts, parts)
    return out[:N]
